# baseline (speedup 1.0000x reference)
import jax
import jax.numpy as jnp
from jax import lax
from jax.experimental import pallas as pl
from jax.experimental.pallas import tpu as pltpu
from jax.experimental.pallas import tpu_sc as plsc

D_MODEL = 1024
MAX_LEN = 2048

NC = 2
NS = 16
NW = NC * NS

BATCH = 4
SEQ = 2048
B = BATCH * SEQ
B_PER_W = B // NW
CHUNK = 32
N_CHUNKS = B_PER_W // CHUNK
NBUF = 3
LANES = 16


def _pe_gather_body(pe_hbm, x_hbm, out_hbm, idx_v, pos_v,
                    buf0, buf1, buf2, sg0, sg1, sg2, ss0, ss1, ss2):
    bufs = (buf0, buf1, buf2)
    sgs = (sg0, sg1, sg2)
    sss = (ss0, ss1, ss2)
    wid = lax.axis_index("s") * NC + lax.axis_index("c")
    base = wid * B_PER_W
    pltpu.sync_copy(x_hbm.at[pl.ds(base, B_PER_W)], idx_v)

    for c in range(N_CHUNKS):
        for g in range(CHUNK // LANES):
            pos_v[c, pl.ds(g * LANES, LANES)] = (
                lax.iota(jnp.int32, LANES) + (base + c * CHUNK + g * LANES))

    def gather(c):
        s = c % NBUF
        src = pe_hbm.at[idx_v.at[pl.ds(c * CHUNK, CHUNK)]]
        return pltpu.make_async_copy(src, bufs[s], sgs[s])

    def scatter(c):
        s = c % NBUF
        dst = out_hbm.at[pos_v.at[c]]
        return pltpu.make_async_copy(bufs[s], dst, sss[s])

    gather(0).start()
    gather(1).start()
    for c in range(N_CHUNKS):
        gather(c).wait()
        scatter(c).start()
        nxt = c + NBUF - 1
        if nxt < N_CHUNKS:
            if c >= 1:
                scatter(c - 1).wait()
            gather(nxt).start()
    scatter(N_CHUNKS - 3).wait()
    scatter(N_CHUNKS - 2).wait()
    scatter(N_CHUNKS - 1).wait()


@jax.jit
def kernel(x, pe):
    mesh = plsc.VectorSubcoreMesh(core_axis_name="c", subcore_axis_name="s")
    run = pl.kernel(
        _pe_gather_body,
        mesh=mesh,
        compiler_params=pltpu.CompilerParams(needs_layout_passes=False),
        out_type=jax.ShapeDtypeStruct((B, 1, D_MODEL), jnp.float32),
        scratch_types=[
            pltpu.VMEM((B_PER_W,), jnp.int32),
            pltpu.VMEM((N_CHUNKS, CHUNK), jnp.int32),
            pltpu.VMEM((CHUNK, 1, D_MODEL), jnp.float32),
            pltpu.VMEM((CHUNK, 1, D_MODEL), jnp.float32),
            pltpu.VMEM((CHUNK, 1, D_MODEL), jnp.float32),
            pltpu.SemaphoreType.DMA,
            pltpu.SemaphoreType.DMA,
            pltpu.SemaphoreType.DMA,
            pltpu.SemaphoreType.DMA,
            pltpu.SemaphoreType.DMA,
            pltpu.SemaphoreType.DMA,
        ],
    )
    out = run(pe, x.reshape(-1).astype(jnp.int32))
    return out.reshape(BATCH, SEQ, 1, D_MODEL)


# speedup vs baseline: 1.0416x; 1.0416x over previous
import jax
import jax.numpy as jnp
from jax import lax
from jax.experimental import pallas as pl
from jax.experimental.pallas import tpu as pltpu
from jax.experimental.pallas import tpu_sc as plsc

D_MODEL = 1024
MAX_LEN = 2048

NC = 2
NS = 16
NW = NC * NS

BATCH = 4
SEQ = 2048
B = BATCH * SEQ
B_PER_W = B // NW
W_PER_ROW = SEQ // B_PER_W
CHUNK = 32
N_CHUNKS = B_PER_W // CHUNK
NBUF = 3


def _pe_gather_body(pe_hbm, x_hbm, out_hbm, idx_v,
                    buf0, buf1, buf2, sg0, sg1, sg2, ss0, ss1, ss2):
    bufs = (buf0, buf1, buf2)
    sgs = (sg0, sg1, sg2)
    sss = (ss0, ss1, ss2)
    wid = lax.axis_index("s") * NC + lax.axis_index("c")
    b = wid // W_PER_ROW
    off = (wid % W_PER_ROW) * B_PER_W
    pltpu.sync_copy(x_hbm.at[b, pl.ds(off, B_PER_W)], idx_v)

    def gather(c):
        s = c % NBUF
        src = pe_hbm.at[idx_v.at[pl.ds(c * CHUNK, CHUNK)]]
        return pltpu.make_async_copy(src, bufs[s], sgs[s])

    def scatter(c):
        s = c % NBUF
        dst = out_hbm.at[b, pl.ds(off + c * CHUNK, CHUNK)]
        return pltpu.make_async_copy(bufs[s], dst, sss[s])

    gather(0).start()
    gather(1).start()
    for c in range(N_CHUNKS):
        gather(c).wait()
        scatter(c).start()
        nxt = c + NBUF - 1
        if nxt < N_CHUNKS:
            if c >= 1:
                scatter(c - 1).wait()
            gather(nxt).start()
    scatter(N_CHUNKS - 3).wait()
    scatter(N_CHUNKS - 2).wait()
    scatter(N_CHUNKS - 1).wait()


@jax.jit
def kernel(x, pe):
    mesh = plsc.VectorSubcoreMesh(core_axis_name="c", subcore_axis_name="s")
    run = pl.kernel(
        _pe_gather_body,
        mesh=mesh,
        out_type=jax.ShapeDtypeStruct((BATCH, SEQ, 1, D_MODEL), jnp.float32),
        scratch_types=[
            pltpu.VMEM((B_PER_W,), jnp.int32),
            pltpu.VMEM((CHUNK, 1, D_MODEL), jnp.float32),
            pltpu.VMEM((CHUNK, 1, D_MODEL), jnp.float32),
            pltpu.VMEM((CHUNK, 1, D_MODEL), jnp.float32),
            pltpu.SemaphoreType.DMA,
            pltpu.SemaphoreType.DMA,
            pltpu.SemaphoreType.DMA,
            pltpu.SemaphoreType.DMA,
            pltpu.SemaphoreType.DMA,
            pltpu.SemaphoreType.DMA,
        ],
    )
    return run(pe, x.astype(jnp.int32))
